# Initial kernel scaffold; baseline (speedup 1.0000x reference)
import jax
import jax.numpy as jnp
from jax.experimental import pallas as pl


def kernel(input, pe_table):
    raise NotImplementedError("write your pallas kernel here")


# speedup vs baseline: 1.8716x; 1.8716x over previous
import jax
import jax.numpy as jnp
from jax.experimental import pallas as pl

_S_BLK = 256


def _add_body(x_ref, pe_ref, o_ref):
    o_ref[...] = x_ref[...] + pe_ref[...][:, None, :]


def kernel(input, pe_table):
    S, B, D = input.shape
    grid = (S // _S_BLK,)
    return pl.pallas_call(
        _add_body,
        grid=grid,
        in_specs=[
            pl.BlockSpec((_S_BLK, B, D), lambda i: (i, 0, 0)),
            pl.BlockSpec((_S_BLK, D), lambda i: (i, 0)),
        ],
        out_specs=pl.BlockSpec((_S_BLK, B, D), lambda i: (i, 0, 0)),
        out_shape=jax.ShapeDtypeStruct((S, B, D), input.dtype),
    )(input, pe_table)


# speedup vs baseline: 1.9095x; 1.0202x over previous
import jax
import jax.numpy as jnp
from jax.experimental import pallas as pl

_S_BLK = 512


def _add_body(x_ref, pe_ref, o_ref):
    o_ref[...] = x_ref[...] + pe_ref[...][:, None, :]


def kernel(input, pe_table):
    S, B, D = input.shape
    grid = (S // _S_BLK,)
    return pl.pallas_call(
        _add_body,
        grid=grid,
        in_specs=[
            pl.BlockSpec((_S_BLK, B, D), lambda i: (i, 0, 0)),
            pl.BlockSpec((_S_BLK, D), lambda i: (i, 0)),
        ],
        out_specs=pl.BlockSpec((_S_BLK, B, D), lambda i: (i, 0, 0)),
        out_shape=jax.ShapeDtypeStruct((S, B, D), input.dtype),
    )(input, pe_table)
